# baseline (speedup 1.0000x reference)
import functools

import jax
import jax.numpy as jnp
from jax import lax
from jax.experimental import pallas as pl
from jax.experimental.pallas import tpu as pltpu
from jax.experimental.pallas import tpu_sc as plsc

B, S, H = 1, 2048, 1024
E, TOPK = 8, 2
DFF = 512
KSZ = 4
T = B * S
TILE = 512
P = T * TOPK + E * TILE
NT = P // TILE
_TSH = 9


def _router_body(x_ref, rw_ref, rb_ref, tri_ref, d0_ref, d1_ref, w0_ref,
                 w1_ref, offs_ref, padded_ref):
    x = x_ref[...]
    logits = jnp.dot(x, rw_ref[...], preferred_element_type=jnp.float32)
    logits = logits + rb_ref[...]
    m = jnp.max(logits, axis=1, keepdims=True)
    ex = jnp.exp(logits - m)
    p = ex / jnp.sum(ex, axis=1, keepdims=True)
    lane = lax.broadcasted_iota(jnp.int32, (T, E), 1)
    v1 = jnp.max(p, axis=1, keepdims=True)
    i1 = jnp.min(jnp.where(p == v1, lane, E), axis=1, keepdims=True)
    m1 = lane == i1
    pm = jnp.where(m1, -1.0, p)
    v2 = jnp.max(pm, axis=1, keepdims=True)
    i2 = jnp.min(jnp.where(pm == v2, lane, E), axis=1, keepdims=True)
    m2 = lane == i2
    s = v1 + v2
    w0_ref[...] = jnp.broadcast_to(v1 / s, (T, 128))
    w1_ref[...] = jnp.broadcast_to(v2 / s, (T, 128))

    cmat = jnp.where(jnp.logical_or(m1, m2), 1.0, 0.0).astype(jnp.bfloat16)
    ccum = jnp.dot(tri_ref[...], cmat, preferred_element_type=jnp.float32)
    rexcl = ccum - cmat.astype(jnp.float32)

    counts = ccum[T - 1:T, :].astype(jnp.int32)
    padded = ((counts + (TILE - 1)) >> _TSH) << _TSH
    er = lax.broadcasted_iota(jnp.int32, (E, E), 0)
    ec = lax.broadcasted_iota(jnp.int32, (E, E), 1)
    strict = jnp.where(er < ec, 1.0, 0.0)
    offs = jnp.dot(padded.astype(jnp.float32), strict,
                   preferred_element_type=jnp.float32)
    offs_b = jnp.broadcast_to(offs, (T, E))
    dest = offs_b + rexcl
    d0_ref[...] = jnp.sum(jnp.where(m1, dest, 0.0), axis=1,
                          keepdims=True).astype(jnp.int32)
    d1_ref[...] = jnp.sum(jnp.where(m2, dest, 0.0), axis=1,
                          keepdims=True).astype(jnp.int32)
    offs_ref[...] = jnp.broadcast_to(offs.astype(jnp.int32), (8, E))
    padded_ref[...] = jnp.broadcast_to(padded, (8, E))


def _expert_body(te_ref, act_ref, xg_ref, sw_ref, wg_ref, wu_ref, wd_ref,
                 out_ref):
    i = pl.program_id(0)

    @pl.when(act_ref[i] == 1)
    def _():
        xb = xg_ref[...].astype(jnp.bfloat16)
        g = jnp.dot(xb, wg_ref[0], preferred_element_type=jnp.float32)
        u = jnp.dot(xb, wu_ref[0], preferred_element_type=jnp.float32)
        h = (g / (1.0 + jnp.exp(-g))) * u
        eo = jnp.dot(h.astype(jnp.bfloat16), wd_ref[0],
                     preferred_element_type=jnp.float32)
        out_ref[...] = eo * sw_ref[:, 0:1]


def _shared_body(xcat_ref, x_ref, wcat_ref, sup_ref, sdn_ref, a_ref, b_ref,
                 out_ref):
    gate = jnp.dot(xcat_ref[...], wcat_ref[...],
                   preferred_element_type=jnp.float32)
    up = jnp.dot(x_ref[...], sup_ref[...], preferred_element_type=jnp.float32)
    h = (gate / (1.0 + jnp.exp(-gate))) * up
    so = jnp.dot(h.astype(jnp.bfloat16), sdn_ref[...],
                 preferred_element_type=jnp.float32)
    out_ref[...] = so + a_ref[...] + b_ref[...]


_NC, _NS = 2, 16
_NW = _NC * _NS
CHUNK = T // _NW
_TPC = T // _NC
_HALF = CHUNK // 2


def _dispatch_sc_body(x_hbm, d0_hbm, d1_hbm, w0_hbm, w1_hbm, xg_hbm, sw_hbm,
                      ia_v, ib_v, rows_v, wa_v, wb_v, sem0, sem1, sem2, sem3):
    cid = lax.axis_index("c")
    sid = lax.axis_index("s")
    wid = sid * _NC + cid
    for h in range(2):
        hb = wid * CHUNK + h * _HALF
        lda = pltpu.async_copy(d0_hbm.at[pl.ds(hb, _HALF)], ia_v, sem0)
        ldb = pltpu.async_copy(d1_hbm.at[pl.ds(hb, _HALF)], ib_v, sem1)
        ldx = pltpu.async_copy(x_hbm.at[pl.ds(hb, _HALF)], rows_v, sem2)
        ldw = pltpu.async_copy(w0_hbm.at[pl.ds(hb, _HALF)], wa_v, sem3)
        lda.wait()
        ldb.wait()
        ldx.wait()
        ldw.wait()
        s0 = pltpu.async_copy(rows_v, xg_hbm.at[ia_v], sem0)
        s1 = pltpu.async_copy(rows_v, xg_hbm.at[ib_v], sem1)
        s2 = pltpu.async_copy(wa_v, sw_hbm.at[ia_v], sem2)
        ldw2 = pltpu.async_copy(w1_hbm.at[pl.ds(hb, _HALF)], wb_v, sem3)
        ldw2.wait()
        s3 = pltpu.async_copy(wb_v, sw_hbm.at[ib_v], sem3)
        s0.wait()
        s1.wait()
        s2.wait()
        s3.wait()


def _gather_sc_body(go_hbm, d0_hbm, d1_hbm, a_hbm, b_hbm,
                    ia_v, ib_v, bufa_v, bufb_v, sem0, sem1):
    cid = lax.axis_index("c")
    sid = lax.axis_index("s")
    for h in range(2):
        hb = (cid * _NS + sid) * CHUNK + h * _HALF
        lda = pltpu.async_copy(d0_hbm.at[pl.ds(hb, _HALF)], ia_v, sem0)
        ldb = pltpu.async_copy(d1_hbm.at[pl.ds(hb, _HALF)], ib_v, sem1)
        lda.wait()
        ldb.wait()
        g0 = pltpu.async_copy(go_hbm.at[ia_v], bufa_v, sem0)
        g1 = pltpu.async_copy(go_hbm.at[ib_v], bufb_v, sem1)
        g0.wait()
        wa = pltpu.async_copy(bufa_v, a_hbm.at[pl.ds(hb, _HALF)], sem0)
        g1.wait()
        wb = pltpu.async_copy(bufb_v, b_hbm.at[pl.ds(hb, _HALF)], sem1)
        wa.wait()
        wb.wait()


@functools.lru_cache(maxsize=None)
def _sc_kernels():
    mesh = plsc.VectorSubcoreMesh(core_axis_name="c", subcore_axis_name="s",
                                  num_cores=_NC, num_subcores=_NS)
    dispatch = pl.kernel(
        _dispatch_sc_body,
        out_type=[jax.ShapeDtypeStruct((P, H), jnp.float32),
                  jax.ShapeDtypeStruct((P, 128), jnp.float32)],
        mesh=mesh,
        scratch_types=[
            pltpu.VMEM((_HALF,), jnp.int32),
            pltpu.VMEM((_HALF,), jnp.int32),
            pltpu.VMEM((_HALF, H), jnp.float32),
            pltpu.VMEM((_HALF, 128), jnp.float32),
            pltpu.VMEM((_HALF, 128), jnp.float32),
            pltpu.SemaphoreType.DMA,
            pltpu.SemaphoreType.DMA,
            pltpu.SemaphoreType.DMA,
            pltpu.SemaphoreType.DMA,
        ],
    )
    gather = pl.kernel(
        _gather_sc_body,
        out_type=[jax.ShapeDtypeStruct((T, H), jnp.float32),
                  jax.ShapeDtypeStruct((T, H), jnp.float32)],
        mesh=mesh,
        scratch_types=[
            pltpu.VMEM((_HALF,), jnp.int32),
            pltpu.VMEM((_HALF,), jnp.int32),
            pltpu.VMEM((_HALF, H), jnp.float32),
            pltpu.VMEM((_HALF, H), jnp.float32),
            pltpu.SemaphoreType.DMA,
            pltpu.SemaphoreType.DMA,
        ],
    )
    return dispatch, gather


def kernel(hidden_states, router_w, router_b, expert_gate, expert_up,
           expert_down, shared_conv_w, shared_up, shared_down):
    x = hidden_states.reshape(T, H)
    x16 = x.astype(jnp.bfloat16)

    d0c, d1c, w0c, w1c, offs8, padded8 = pl.pallas_call(
        _router_body,
        out_shape=[
            jax.ShapeDtypeStruct((T, 1), jnp.int32),
            jax.ShapeDtypeStruct((T, 1), jnp.int32),
            jax.ShapeDtypeStruct((T, 128), jnp.float32),
            jax.ShapeDtypeStruct((T, 128), jnp.float32),
            jax.ShapeDtypeStruct((8, E), jnp.int32),
            jax.ShapeDtypeStruct((8, E), jnp.int32),
        ],
    )(x16, router_w.astype(jnp.bfloat16), router_b.reshape(1, E),
      jnp.tril(jnp.ones((T, T), jnp.bfloat16)))

    d0 = d0c.reshape(T)
    d1 = d1c.reshape(T)

    offs = offs8[0]
    padded = padded8[0]
    total = offs[E - 1] + padded[E - 1]
    tile_start = jnp.arange(NT, dtype=jnp.int32) * TILE
    te_idx = jnp.sum(
        (tile_start[:, None] >= offs[None, :]).astype(jnp.int32), axis=1) - 1
    act = (tile_start < total).astype(jnp.int32)
    last_te = te_idx[jnp.maximum((total >> _TSH) - 1, 0)]
    te_fix = jnp.where(act == 1, te_idx, last_te).astype(jnp.int32)

    _dispatch_sc, _gather_sc = _sc_kernels()
    xg, sw = _dispatch_sc(x, d0, d1, w0c, w1c)

    go = pl.pallas_call(
        _expert_body,
        grid_spec=pltpu.PrefetchScalarGridSpec(
            num_scalar_prefetch=2,
            grid=(NT,),
            in_specs=[
                pl.BlockSpec((TILE, H), lambda i, te, a: (i, 0)),
                pl.BlockSpec((TILE, 128), lambda i, te, a: (i, 0)),
                pl.BlockSpec((1, H, DFF), lambda i, te, a: (te[i], 0, 0)),
                pl.BlockSpec((1, H, DFF), lambda i, te, a: (te[i], 0, 0)),
                pl.BlockSpec((1, DFF, H), lambda i, te, a: (te[i], 0, 0)),
            ],
            out_specs=pl.BlockSpec((TILE, H), lambda i, te, a: (i, 0)),
        ),
        out_shape=jax.ShapeDtypeStruct((P, H), jnp.float32),
    )(te_fix, act, xg, sw,
      expert_gate.astype(jnp.bfloat16),
      expert_up.astype(jnp.bfloat16),
      expert_down.astype(jnp.bfloat16))

    arows, brows = _gather_sc(go, d0, d1)

    xp = jnp.pad(x16, ((KSZ - 1, 0), (0, 0)))
    xcat = jnp.concatenate([xp[k:T + k] for k in range(KSZ)], axis=1)
    wcat = jnp.concatenate(
        [shared_conv_w[:, :, k].T for k in range(KSZ)], axis=0
    ).astype(jnp.bfloat16)
    TM = 512
    out = pl.pallas_call(
        _shared_body,
        grid=(T // TM,),
        in_specs=[
            pl.BlockSpec((TM, KSZ * H), lambda i: (i, 0)),
            pl.BlockSpec((TM, H), lambda i: (i, 0)),
            pl.BlockSpec((KSZ * H, DFF), lambda i: (0, 0)),
            pl.BlockSpec((H, DFF), lambda i: (0, 0)),
            pl.BlockSpec((DFF, H), lambda i: (0, 0)),
            pl.BlockSpec((TM, H), lambda i: (i, 0)),
            pl.BlockSpec((TM, H), lambda i: (i, 0)),
        ],
        out_specs=pl.BlockSpec((TM, H), lambda i: (i, 0)),
        out_shape=jax.ShapeDtypeStruct((T, H), jnp.float32),
    )(xcat, x16, wcat,
      shared_up.astype(jnp.bfloat16), shared_down.astype(jnp.bfloat16),
      arows, brows)
    return out.reshape(B, S, H)


# speedup vs baseline: 1.0222x; 1.0222x over previous
import functools

import jax
import jax.numpy as jnp
from jax import lax
from jax.experimental import pallas as pl
from jax.experimental.pallas import tpu as pltpu
from jax.experimental.pallas import tpu_sc as plsc

B, S, H = 1, 2048, 1024
E, TOPK = 8, 2
DFF = 512
KSZ = 4
T = B * S
TILE = 256
P = T * TOPK + E * TILE
NT = P // TILE
_TSH = 8


def _router_body(x_ref, rw_ref, rb_ref, d0_ref, d1_ref, w0_ref,
                 w1_ref, offs_ref, padded_ref):
    x = x_ref[...]
    logits = jnp.dot(x, rw_ref[...], preferred_element_type=jnp.float32)
    logits = logits + rb_ref[...]
    m = jnp.max(logits, axis=1, keepdims=True)
    ex = jnp.exp(logits - m)
    p = ex / jnp.sum(ex, axis=1, keepdims=True)
    lane = lax.broadcasted_iota(jnp.int32, (T, E), 1)
    v1 = jnp.max(p, axis=1, keepdims=True)
    i1 = jnp.min(jnp.where(p == v1, lane, E), axis=1, keepdims=True)
    m1 = lane == i1
    pm = jnp.where(m1, -1.0, p)
    v2 = jnp.max(pm, axis=1, keepdims=True)
    i2 = jnp.min(jnp.where(pm == v2, lane, E), axis=1, keepdims=True)
    m2 = lane == i2
    s = v1 + v2
    w0_ref[...] = jnp.broadcast_to(v1 / s, (T, 128))
    w1_ref[...] = jnp.broadcast_to(v2 / s, (T, 128))

    cmat = jnp.where(jnp.logical_or(m1, m2), 1.0, 0.0).astype(jnp.bfloat16)
    SEG = 128
    r = lax.broadcasted_iota(jnp.int32, (SEG, SEG), 0)
    c = lax.broadcasted_iota(jnp.int32, (SEG, SEG), 1)
    ltri = jnp.where(r >= c, 1.0, 0.0).astype(jnp.bfloat16)
    segs = []
    run = jnp.zeros((1, E), jnp.float32)
    for i in range(T // SEG):
        seg = cmat[i * SEG:(i + 1) * SEG]
        scum = jnp.dot(ltri, seg, preferred_element_type=jnp.float32) + run
        segs.append(scum)
        run = scum[SEG - 1:SEG, :]
    ccum = jnp.concatenate(segs, axis=0)
    rexcl = ccum - cmat.astype(jnp.float32)

    counts = run.astype(jnp.int32)
    padded = ((counts + (TILE - 1)) >> _TSH) << _TSH
    er = lax.broadcasted_iota(jnp.int32, (E, E), 0)
    ec = lax.broadcasted_iota(jnp.int32, (E, E), 1)
    strict = jnp.where(er < ec, 1.0, 0.0)
    offs = jnp.dot(padded.astype(jnp.float32), strict,
                   preferred_element_type=jnp.float32)
    offs_b = jnp.broadcast_to(offs, (T, E))
    dest = offs_b + rexcl
    d0_ref[...] = jnp.sum(jnp.where(m1, dest, 0.0), axis=1,
                          keepdims=True).astype(jnp.int32)
    d1_ref[...] = jnp.sum(jnp.where(m2, dest, 0.0), axis=1,
                          keepdims=True).astype(jnp.int32)
    offs_ref[...] = jnp.broadcast_to(offs.astype(jnp.int32), (8, E))
    padded_ref[...] = jnp.broadcast_to(padded, (8, E))


def _expert_body(te_ref, act_ref, xg_ref, sw_ref, wg_ref, wu_ref, wd_ref,
                 out_ref):
    i = pl.program_id(0)

    @pl.when(act_ref[i] == 1)
    def _():
        xb = xg_ref[...].astype(jnp.bfloat16)
        g = jnp.dot(xb, wg_ref[0], preferred_element_type=jnp.float32)
        u = jnp.dot(xb, wu_ref[0], preferred_element_type=jnp.float32)
        h = (g / (1.0 + jnp.exp(-g))) * u
        eo = jnp.dot(h.astype(jnp.bfloat16), wd_ref[0],
                     preferred_element_type=jnp.float32)
        out_ref[...] = eo * sw_ref[:, 0:1]


def _shared_body(xcat_ref, x_ref, wcat_ref, sup_ref, sdn_ref, a_ref, b_ref,
                 out_ref):
    gate = jnp.dot(xcat_ref[...], wcat_ref[...],
                   preferred_element_type=jnp.float32)
    up = jnp.dot(x_ref[...], sup_ref[...], preferred_element_type=jnp.float32)
    h = (gate / (1.0 + jnp.exp(-gate))) * up
    so = jnp.dot(h.astype(jnp.bfloat16), sdn_ref[...],
                 preferred_element_type=jnp.float32)
    out_ref[...] = so + a_ref[...] + b_ref[...]


_NC, _NS = 2, 16
_NW = _NC * _NS
CHUNK = T // _NW
_TPC = T // _NC
_HALF = CHUNK // 2


def _dispatch_sc_body(x_hbm, d0_hbm, d1_hbm, w0_hbm, w1_hbm, xg_hbm, sw_hbm,
                      ia_v, ib_v, rows_v, wa_v, wb_v, sem0, sem1, sem2, sem3):
    cid = lax.axis_index("c")
    sid = lax.axis_index("s")
    wid = sid * _NC + cid
    for h in range(2):
        hb = wid * CHUNK + h * _HALF
        lda = pltpu.async_copy(d0_hbm.at[pl.ds(hb, _HALF)], ia_v, sem0)
        ldb = pltpu.async_copy(d1_hbm.at[pl.ds(hb, _HALF)], ib_v, sem1)
        ldx = pltpu.async_copy(x_hbm.at[pl.ds(hb, _HALF)], rows_v, sem2)
        ldw = pltpu.async_copy(w0_hbm.at[pl.ds(hb, _HALF)], wa_v, sem3)
        lda.wait()
        ldb.wait()
        ldx.wait()
        ldw.wait()
        s0 = pltpu.async_copy(rows_v, xg_hbm.at[ia_v], sem0)
        s1 = pltpu.async_copy(rows_v, xg_hbm.at[ib_v], sem1)
        s2 = pltpu.async_copy(wa_v, sw_hbm.at[ia_v], sem2)
        ldw2 = pltpu.async_copy(w1_hbm.at[pl.ds(hb, _HALF)], wb_v, sem3)
        ldw2.wait()
        s3 = pltpu.async_copy(wb_v, sw_hbm.at[ib_v], sem3)
        s0.wait()
        s1.wait()
        s2.wait()
        s3.wait()


def _gather_sc_body(go_hbm, d0_hbm, d1_hbm, a_hbm, b_hbm,
                    ia_v, ib_v, bufa_v, bufb_v, sem0, sem1):
    cid = lax.axis_index("c")
    sid = lax.axis_index("s")
    for h in range(2):
        hb = (cid * _NS + sid) * CHUNK + h * _HALF
        lda = pltpu.async_copy(d0_hbm.at[pl.ds(hb, _HALF)], ia_v, sem0)
        ldb = pltpu.async_copy(d1_hbm.at[pl.ds(hb, _HALF)], ib_v, sem1)
        lda.wait()
        ldb.wait()
        g0 = pltpu.async_copy(go_hbm.at[ia_v], bufa_v, sem0)
        g1 = pltpu.async_copy(go_hbm.at[ib_v], bufb_v, sem1)
        g0.wait()
        wa = pltpu.async_copy(bufa_v, a_hbm.at[pl.ds(hb, _HALF)], sem0)
        g1.wait()
        wb = pltpu.async_copy(bufb_v, b_hbm.at[pl.ds(hb, _HALF)], sem1)
        wa.wait()
        wb.wait()


@functools.lru_cache(maxsize=None)
def _sc_kernels():
    mesh = plsc.VectorSubcoreMesh(core_axis_name="c", subcore_axis_name="s",
                                  num_cores=_NC, num_subcores=_NS)
    dispatch = pl.kernel(
        _dispatch_sc_body,
        out_type=[jax.ShapeDtypeStruct((P, H), jnp.float32),
                  jax.ShapeDtypeStruct((P, 128), jnp.float32)],
        mesh=mesh,
        scratch_types=[
            pltpu.VMEM((_HALF,), jnp.int32),
            pltpu.VMEM((_HALF,), jnp.int32),
            pltpu.VMEM((_HALF, H), jnp.float32),
            pltpu.VMEM((_HALF, 128), jnp.float32),
            pltpu.VMEM((_HALF, 128), jnp.float32),
            pltpu.SemaphoreType.DMA,
            pltpu.SemaphoreType.DMA,
            pltpu.SemaphoreType.DMA,
            pltpu.SemaphoreType.DMA,
        ],
    )
    gather = pl.kernel(
        _gather_sc_body,
        out_type=[jax.ShapeDtypeStruct((T, H), jnp.float32),
                  jax.ShapeDtypeStruct((T, H), jnp.float32)],
        mesh=mesh,
        scratch_types=[
            pltpu.VMEM((_HALF,), jnp.int32),
            pltpu.VMEM((_HALF,), jnp.int32),
            pltpu.VMEM((_HALF, H), jnp.float32),
            pltpu.VMEM((_HALF, H), jnp.float32),
            pltpu.SemaphoreType.DMA,
            pltpu.SemaphoreType.DMA,
        ],
    )
    return dispatch, gather


def kernel(hidden_states, router_w, router_b, expert_gate, expert_up,
           expert_down, shared_conv_w, shared_up, shared_down):
    x = hidden_states.reshape(T, H)
    x16 = x.astype(jnp.bfloat16)

    d0c, d1c, w0c, w1c, offs8, padded8 = pl.pallas_call(
        _router_body,
        out_shape=[
            jax.ShapeDtypeStruct((T, 1), jnp.int32),
            jax.ShapeDtypeStruct((T, 1), jnp.int32),
            jax.ShapeDtypeStruct((T, 128), jnp.float32),
            jax.ShapeDtypeStruct((T, 128), jnp.float32),
            jax.ShapeDtypeStruct((8, E), jnp.int32),
            jax.ShapeDtypeStruct((8, E), jnp.int32),
        ],
    )(x16, router_w.astype(jnp.bfloat16), router_b.reshape(1, E))

    d0 = d0c.reshape(T)
    d1 = d1c.reshape(T)

    offs = offs8[0]
    padded = padded8[0]
    total = offs[E - 1] + padded[E - 1]
    tile_start = jnp.arange(NT, dtype=jnp.int32) * TILE
    te_idx = jnp.sum(
        (tile_start[:, None] >= offs[None, :]).astype(jnp.int32), axis=1) - 1
    act = (tile_start < total).astype(jnp.int32)
    last_te = te_idx[jnp.maximum((total >> _TSH) - 1, 0)]
    te_fix = jnp.where(act == 1, te_idx, last_te).astype(jnp.int32)

    _dispatch_sc, _gather_sc = _sc_kernels()
    xg, sw = _dispatch_sc(x, d0, d1, w0c, w1c)

    go = pl.pallas_call(
        _expert_body,
        grid_spec=pltpu.PrefetchScalarGridSpec(
            num_scalar_prefetch=2,
            grid=(NT,),
            in_specs=[
                pl.BlockSpec((TILE, H), lambda i, te, a: (i, 0)),
                pl.BlockSpec((TILE, 128), lambda i, te, a: (i, 0)),
                pl.BlockSpec((1, H, DFF), lambda i, te, a: (te[i], 0, 0)),
                pl.BlockSpec((1, H, DFF), lambda i, te, a: (te[i], 0, 0)),
                pl.BlockSpec((1, DFF, H), lambda i, te, a: (te[i], 0, 0)),
            ],
            out_specs=pl.BlockSpec((TILE, H), lambda i, te, a: (i, 0)),
        ),
        out_shape=jax.ShapeDtypeStruct((P, H), jnp.float32),
    )(te_fix, act, xg, sw,
      expert_gate.astype(jnp.bfloat16),
      expert_up.astype(jnp.bfloat16),
      expert_down.astype(jnp.bfloat16))

    arows, brows = _gather_sc(go, d0, d1)

    xp = jnp.pad(x16, ((KSZ - 1, 0), (0, 0)))
    xcat = jnp.concatenate([xp[k:T + k] for k in range(KSZ)], axis=1)
    wcat = jnp.concatenate(
        [shared_conv_w[:, :, k].T for k in range(KSZ)], axis=0
    ).astype(jnp.bfloat16)
    TM = 512
    out = pl.pallas_call(
        _shared_body,
        grid=(T // TM,),
        in_specs=[
            pl.BlockSpec((TM, KSZ * H), lambda i: (i, 0)),
            pl.BlockSpec((TM, H), lambda i: (i, 0)),
            pl.BlockSpec((KSZ * H, DFF), lambda i: (0, 0)),
            pl.BlockSpec((H, DFF), lambda i: (0, 0)),
            pl.BlockSpec((DFF, H), lambda i: (0, 0)),
            pl.BlockSpec((TM, H), lambda i: (i, 0)),
            pl.BlockSpec((TM, H), lambda i: (i, 0)),
        ],
        out_specs=pl.BlockSpec((TM, H), lambda i: (i, 0)),
        out_shape=jax.ShapeDtypeStruct((T, H), jnp.float32),
    )(xcat, x16, wcat,
      shared_up.astype(jnp.bfloat16), shared_down.astype(jnp.bfloat16),
      arows, brows)
    return out.reshape(B, S, H)


# speedup vs baseline: 1.0732x; 1.0500x over previous
import functools

import jax
import jax.numpy as jnp
from jax import lax
from jax.experimental import pallas as pl
from jax.experimental.pallas import tpu as pltpu
from jax.experimental.pallas import tpu_sc as plsc

B, S, H = 1, 2048, 1024
E, TOPK = 8, 2
DFF = 512
KSZ = 4
T = B * S
TILE = 512
P = T * TOPK + E * TILE
NT = P // TILE
_TSH = 9


def _router_body(x_ref, rw_ref, rb_ref, d0_ref, d1_ref, w0_ref,
                 w1_ref, offs_ref, padded_ref):
    x = x_ref[...]
    logits = jnp.dot(x, rw_ref[...], preferred_element_type=jnp.float32)
    logits = logits + rb_ref[...]
    m = jnp.max(logits, axis=1, keepdims=True)
    ex = jnp.exp(logits - m)
    p = ex / jnp.sum(ex, axis=1, keepdims=True)
    lane = lax.broadcasted_iota(jnp.int32, (T, E), 1)
    v1 = jnp.max(p, axis=1, keepdims=True)
    i1 = jnp.min(jnp.where(p == v1, lane, E), axis=1, keepdims=True)
    m1 = lane == i1
    pm = jnp.where(m1, -1.0, p)
    v2 = jnp.max(pm, axis=1, keepdims=True)
    i2 = jnp.min(jnp.where(pm == v2, lane, E), axis=1, keepdims=True)
    m2 = lane == i2
    s = v1 + v2
    w0_ref[...] = v1 / s
    w1_ref[...] = v2 / s

    cmat = jnp.where(jnp.logical_or(m1, m2), 1.0, 0.0).astype(jnp.bfloat16)
    SEG = 128
    r = lax.broadcasted_iota(jnp.int32, (SEG, SEG), 0)
    c = lax.broadcasted_iota(jnp.int32, (SEG, SEG), 1)
    ltri = jnp.where(r >= c, 1.0, 0.0).astype(jnp.bfloat16)
    segs = []
    run = jnp.zeros((1, E), jnp.float32)
    for i in range(T // SEG):
        seg = cmat[i * SEG:(i + 1) * SEG]
        scum = jnp.dot(ltri, seg, preferred_element_type=jnp.float32) + run
        segs.append(scum)
        run = scum[SEG - 1:SEG, :]
    ccum = jnp.concatenate(segs, axis=0)
    rexcl = ccum - cmat.astype(jnp.float32)

    counts = run.astype(jnp.int32)
    padded = ((counts + (TILE - 1)) >> _TSH) << _TSH
    er = lax.broadcasted_iota(jnp.int32, (E, E), 0)
    ec = lax.broadcasted_iota(jnp.int32, (E, E), 1)
    strict = jnp.where(er < ec, 1.0, 0.0)
    offs = jnp.dot(padded.astype(jnp.float32), strict,
                   preferred_element_type=jnp.float32)
    offs_b = jnp.broadcast_to(offs, (T, E))
    dest = offs_b + rexcl
    d0_ref[...] = jnp.sum(jnp.where(m1, dest, 0.0), axis=1,
                          keepdims=True).astype(jnp.int32)
    d1_ref[...] = jnp.sum(jnp.where(m2, dest, 0.0), axis=1,
                          keepdims=True).astype(jnp.int32)
    offs_ref[...] = jnp.broadcast_to(offs.astype(jnp.int32), (8, E))
    padded_ref[...] = jnp.broadcast_to(padded, (8, E))


def _expert_body(te_ref, act_ref, xg_ref, wg_ref, wu_ref, wd_ref, out_ref):
    i = pl.program_id(0)

    @pl.when(act_ref[i] == 1)
    def _():
        xb = xg_ref[...].astype(jnp.bfloat16)
        g = jnp.dot(xb, wg_ref[0], preferred_element_type=jnp.float32)
        u = jnp.dot(xb, wu_ref[0], preferred_element_type=jnp.float32)
        h = (g / (1.0 + jnp.exp(-g))) * u
        out_ref[...] = jnp.dot(h.astype(jnp.bfloat16), wd_ref[0],
                               preferred_element_type=jnp.float32)


def _shared_body(xcat_ref, x_ref, wcat_ref, sup_ref, sdn_ref, a_ref, b_ref,
                 w0_ref, w1_ref, out_ref):
    gate = jnp.dot(xcat_ref[...], wcat_ref[...],
                   preferred_element_type=jnp.float32)
    up = jnp.dot(x_ref[...], sup_ref[...], preferred_element_type=jnp.float32)
    h = (gate / (1.0 + jnp.exp(-gate))) * up
    so = jnp.dot(h.astype(jnp.bfloat16), sdn_ref[...],
                 preferred_element_type=jnp.float32)
    out_ref[...] = so + w0_ref[...] * a_ref[...] + w1_ref[...] * b_ref[...]


_NC, _NS = 2, 16
_NW = _NC * _NS
CHUNK = T // _NW
_TPC = T // _NC
_HALF = CHUNK // 2


def _dispatch_sc_body(x_hbm, d0_hbm, d1_hbm, xg_hbm,
                      ia_v, ib_v, rows_v, sem0, sem1, sem2):
    cid = lax.axis_index("c")
    sid = lax.axis_index("s")
    wid = sid * _NC + cid
    for h in range(2):
        hb = wid * CHUNK + h * _HALF
        lda = pltpu.async_copy(d0_hbm.at[pl.ds(hb, _HALF)], ia_v, sem0)
        ldb = pltpu.async_copy(d1_hbm.at[pl.ds(hb, _HALF)], ib_v, sem1)
        ldx = pltpu.async_copy(x_hbm.at[pl.ds(hb, _HALF)], rows_v, sem2)
        lda.wait()
        ldb.wait()
        ldx.wait()
        s0 = pltpu.async_copy(rows_v, xg_hbm.at[ia_v], sem0)
        s1 = pltpu.async_copy(rows_v, xg_hbm.at[ib_v], sem1)
        s0.wait()
        s1.wait()


def _gather_sc_body(go_hbm, d0_hbm, d1_hbm, a_hbm, b_hbm,
                    ia_v, ib_v, bufa_v, bufb_v, sem0, sem1):
    cid = lax.axis_index("c")
    sid = lax.axis_index("s")
    for h in range(2):
        hb = (cid * _NS + sid) * CHUNK + h * _HALF
        lda = pltpu.async_copy(d0_hbm.at[pl.ds(hb, _HALF)], ia_v, sem0)
        ldb = pltpu.async_copy(d1_hbm.at[pl.ds(hb, _HALF)], ib_v, sem1)
        lda.wait()
        ldb.wait()
        g0 = pltpu.async_copy(go_hbm.at[ia_v], bufa_v, sem0)
        g1 = pltpu.async_copy(go_hbm.at[ib_v], bufb_v, sem1)
        g0.wait()
        wa = pltpu.async_copy(bufa_v, a_hbm.at[pl.ds(hb, _HALF)], sem0)
        g1.wait()
        wb = pltpu.async_copy(bufb_v, b_hbm.at[pl.ds(hb, _HALF)], sem1)
        wa.wait()
        wb.wait()


@functools.lru_cache(maxsize=None)
def _sc_kernels():
    mesh = plsc.VectorSubcoreMesh(core_axis_name="c", subcore_axis_name="s",
                                  num_cores=_NC, num_subcores=_NS)
    dispatch = pl.kernel(
        _dispatch_sc_body,
        out_type=jax.ShapeDtypeStruct((P, H), jnp.float32),
        mesh=mesh,
        scratch_types=[
            pltpu.VMEM((_HALF,), jnp.int32),
            pltpu.VMEM((_HALF,), jnp.int32),
            pltpu.VMEM((_HALF, H), jnp.float32),
            pltpu.SemaphoreType.DMA,
            pltpu.SemaphoreType.DMA,
            pltpu.SemaphoreType.DMA,
        ],
    )
    gather = pl.kernel(
        _gather_sc_body,
        out_type=[jax.ShapeDtypeStruct((T, H), jnp.float32),
                  jax.ShapeDtypeStruct((T, H), jnp.float32)],
        mesh=mesh,
        scratch_types=[
            pltpu.VMEM((_HALF,), jnp.int32),
            pltpu.VMEM((_HALF,), jnp.int32),
            pltpu.VMEM((_HALF, H), jnp.float32),
            pltpu.VMEM((_HALF, H), jnp.float32),
            pltpu.SemaphoreType.DMA,
            pltpu.SemaphoreType.DMA,
        ],
    )
    return dispatch, gather


def kernel(hidden_states, router_w, router_b, expert_gate, expert_up,
           expert_down, shared_conv_w, shared_up, shared_down):
    x = hidden_states.reshape(T, H)
    x16 = x.astype(jnp.bfloat16)

    d0c, d1c, w0c, w1c, offs8, padded8 = pl.pallas_call(
        _router_body,
        out_shape=[
            jax.ShapeDtypeStruct((T, 1), jnp.int32),
            jax.ShapeDtypeStruct((T, 1), jnp.int32),
            jax.ShapeDtypeStruct((T, 1), jnp.float32),
            jax.ShapeDtypeStruct((T, 1), jnp.float32),
            jax.ShapeDtypeStruct((8, E), jnp.int32),
            jax.ShapeDtypeStruct((8, E), jnp.int32),
        ],
    )(x16, router_w.astype(jnp.bfloat16), router_b.reshape(1, E))

    d0 = d0c.reshape(T)
    d1 = d1c.reshape(T)

    offs = offs8[0]
    padded = padded8[0]
    total = offs[E - 1] + padded[E - 1]
    tile_start = jnp.arange(NT, dtype=jnp.int32) * TILE
    te_idx = jnp.sum(
        (tile_start[:, None] >= offs[None, :]).astype(jnp.int32), axis=1) - 1
    act = (tile_start < total).astype(jnp.int32)
    last_te = te_idx[jnp.maximum((total >> _TSH) - 1, 0)]
    te_fix = jnp.where(act == 1, te_idx, last_te).astype(jnp.int32)

    _dispatch_sc, _gather_sc = _sc_kernels()
    xg = _dispatch_sc(x, d0, d1)

    go = pl.pallas_call(
        _expert_body,
        grid_spec=pltpu.PrefetchScalarGridSpec(
            num_scalar_prefetch=2,
            grid=(NT,),
            in_specs=[
                pl.BlockSpec(
                    (TILE, H),
                    lambda i, te, a: (jnp.where(a[i] == 1, i, 0), 0)),
                pl.BlockSpec((1, H, DFF), lambda i, te, a: (te[i], 0, 0)),
                pl.BlockSpec((1, H, DFF), lambda i, te, a: (te[i], 0, 0)),
                pl.BlockSpec((1, DFF, H), lambda i, te, a: (te[i], 0, 0)),
            ],
            out_specs=pl.BlockSpec(
                (TILE, H),
                lambda i, te, a: (jnp.where(a[i] == 1, i, NT - 1), 0)),
        ),
        out_shape=jax.ShapeDtypeStruct((P, H), jnp.float32),
    )(te_fix, act, xg,
      expert_gate.astype(jnp.bfloat16),
      expert_up.astype(jnp.bfloat16),
      expert_down.astype(jnp.bfloat16))

    arows, brows = _gather_sc(go, d0, d1)

    xp = jnp.pad(x16, ((KSZ - 1, 0), (0, 0)))
    xcat = jnp.concatenate([xp[k:T + k] for k in range(KSZ)], axis=1)
    wcat = jnp.concatenate(
        [shared_conv_w[:, :, k].T for k in range(KSZ)], axis=0
    ).astype(jnp.bfloat16)
    TM = 512
    out = pl.pallas_call(
        _shared_body,
        grid=(T // TM,),
        in_specs=[
            pl.BlockSpec((TM, KSZ * H), lambda i: (i, 0)),
            pl.BlockSpec((TM, H), lambda i: (i, 0)),
            pl.BlockSpec((KSZ * H, DFF), lambda i: (0, 0)),
            pl.BlockSpec((H, DFF), lambda i: (0, 0)),
            pl.BlockSpec((DFF, H), lambda i: (0, 0)),
            pl.BlockSpec((TM, H), lambda i: (i, 0)),
            pl.BlockSpec((TM, H), lambda i: (i, 0)),
            pl.BlockSpec((TM, 1), lambda i: (i, 0)),
            pl.BlockSpec((TM, 1), lambda i: (i, 0)),
        ],
        out_specs=pl.BlockSpec((TM, H), lambda i: (i, 0)),
        out_shape=jax.ShapeDtypeStruct((T, H), jnp.float32),
    )(xcat, x16, wcat,
      shared_up.astype(jnp.bfloat16), shared_down.astype(jnp.bfloat16),
      arows, brows, w0c, w1c)
    return out.reshape(B, S, H)


# speedup vs baseline: 1.0832x; 1.0093x over previous
import functools

import jax
import jax.numpy as jnp
from jax import lax
from jax.experimental import pallas as pl
from jax.experimental.pallas import tpu as pltpu
from jax.experimental.pallas import tpu_sc as plsc

B, S, H = 1, 2048, 1024
E, TOPK = 8, 2
DFF = 512
KSZ = 4
T = B * S
TILE = 512
P = T * TOPK + E * TILE
NT = P // TILE
_TSH = 9


def _router_body(x_ref, rw_ref, rb_ref, d0_ref, d1_ref, w0_ref,
                 w1_ref, offs_ref, padded_ref):
    x = x_ref[...]
    logits = jnp.dot(x, rw_ref[...], preferred_element_type=jnp.float32)
    logits = logits + rb_ref[...]
    m = jnp.max(logits, axis=1, keepdims=True)
    ex = jnp.exp(logits - m)
    p = ex / jnp.sum(ex, axis=1, keepdims=True)
    lane = lax.broadcasted_iota(jnp.int32, (T, E), 1)
    v1 = jnp.max(p, axis=1, keepdims=True)
    i1 = jnp.min(jnp.where(p == v1, lane, E), axis=1, keepdims=True)
    m1 = lane == i1
    pm = jnp.where(m1, -1.0, p)
    v2 = jnp.max(pm, axis=1, keepdims=True)
    i2 = jnp.min(jnp.where(pm == v2, lane, E), axis=1, keepdims=True)
    m2 = lane == i2
    s = v1 + v2
    w0_ref[...] = v1 / s
    w1_ref[...] = v2 / s

    cmat = jnp.where(jnp.logical_or(m1, m2), 1.0, 0.0).astype(jnp.bfloat16)
    SEG = 128
    r = lax.broadcasted_iota(jnp.int32, (SEG, SEG), 0)
    c = lax.broadcasted_iota(jnp.int32, (SEG, SEG), 1)
    ltri = jnp.where(r >= c, 1.0, 0.0).astype(jnp.bfloat16)
    segs = []
    run = jnp.zeros((1, E), jnp.float32)
    for i in range(T // SEG):
        seg = cmat[i * SEG:(i + 1) * SEG]
        scum = jnp.dot(ltri, seg, preferred_element_type=jnp.float32) + run
        segs.append(scum)
        run = scum[SEG - 1:SEG, :]
    ccum = jnp.concatenate(segs, axis=0)
    rexcl = ccum - cmat.astype(jnp.float32)

    counts = run.astype(jnp.int32)
    padded = ((counts + (TILE - 1)) >> _TSH) << _TSH
    er = lax.broadcasted_iota(jnp.int32, (E, E), 0)
    ec = lax.broadcasted_iota(jnp.int32, (E, E), 1)
    strict = jnp.where(er < ec, 1.0, 0.0)
    offs = jnp.dot(padded.astype(jnp.float32), strict,
                   preferred_element_type=jnp.float32)
    offs_b = jnp.broadcast_to(offs, (T, E))
    dest = offs_b + rexcl
    d0_ref[...] = jnp.sum(jnp.where(m1, dest, 0.0), axis=1,
                          keepdims=True).astype(jnp.int32)
    d1_ref[...] = jnp.sum(jnp.where(m2, dest, 0.0), axis=1,
                          keepdims=True).astype(jnp.int32)
    offs_ref[...] = jnp.broadcast_to(offs.astype(jnp.int32), (8, E))
    padded_ref[...] = jnp.broadcast_to(padded, (8, E))


def _expert_body(te_ref, act_ref, xg_ref, wg_ref, wu_ref, wd_ref, out_ref):
    i = pl.program_id(0)

    @pl.when(act_ref[i] == 1)
    def _():
        xb = xg_ref[...].astype(jnp.bfloat16)
        g = jnp.dot(xb, wg_ref[0], preferred_element_type=jnp.float32)
        u = jnp.dot(xb, wu_ref[0], preferred_element_type=jnp.float32)
        h = (g / (1.0 + jnp.exp(-g))) * u
        out_ref[...] = jnp.dot(h.astype(jnp.bfloat16), wd_ref[0],
                               preferred_element_type=jnp.float32)


def _shared_body(xcat_ref, x_ref, wcat_ref, sup_ref, sdn_ref, a_ref, b_ref,
                 w0_ref, w1_ref, out_ref):
    gate = jnp.dot(xcat_ref[...], wcat_ref[...],
                   preferred_element_type=jnp.float32)
    up = jnp.dot(x_ref[...], sup_ref[...], preferred_element_type=jnp.float32)
    h = (gate / (1.0 + jnp.exp(-gate))) * up
    so = jnp.dot(h.astype(jnp.bfloat16), sdn_ref[...],
                 preferred_element_type=jnp.float32)
    out_ref[...] = so + w0_ref[...] * a_ref[...] + w1_ref[...] * b_ref[...]


_NC, _NS = 2, 16
_NW = _NC * _NS
CHUNK = T // _NW
_TPC = T // _NC
_HALF = CHUNK // 2


def _dispatch_sc_body(x_hbm, d0_hbm, d1_hbm, xg_hbm,
                      ia_v, ib_v, rows_v, sem0, sem1, sem2):
    cid = lax.axis_index("c")
    sid = lax.axis_index("s")
    base = (sid * _NC + cid) * CHUNK
    lda = pltpu.async_copy(d0_hbm.at[pl.ds(base, CHUNK)], ia_v, sem0)
    ldb = pltpu.async_copy(d1_hbm.at[pl.ds(base, CHUNK)], ib_v, sem1)
    ldx = pltpu.async_copy(x_hbm.at[pl.ds(base, CHUNK)], rows_v, sem2)
    lda.wait()
    ldb.wait()
    ldx.wait()
    s0 = pltpu.async_copy(rows_v, xg_hbm.at[ia_v], sem0)
    s1 = pltpu.async_copy(rows_v, xg_hbm.at[ib_v], sem1)
    s0.wait()
    s1.wait()


def _gather_sc_body(go_hbm, d0_hbm, d1_hbm, a_hbm, b_hbm,
                    ia_v, ib_v, bufa_v, bufb_v, sem0, sem1):
    cid = lax.axis_index("c")
    sid = lax.axis_index("s")
    for h in range(2):
        hb = (cid * _NS + sid) * CHUNK + h * _HALF
        lda = pltpu.async_copy(d0_hbm.at[pl.ds(hb, _HALF)], ia_v, sem0)
        ldb = pltpu.async_copy(d1_hbm.at[pl.ds(hb, _HALF)], ib_v, sem1)
        lda.wait()
        ldb.wait()
        g0 = pltpu.async_copy(go_hbm.at[ia_v], bufa_v, sem0)
        g1 = pltpu.async_copy(go_hbm.at[ib_v], bufb_v, sem1)
        g0.wait()
        wa = pltpu.async_copy(bufa_v, a_hbm.at[pl.ds(hb, _HALF)], sem0)
        g1.wait()
        wb = pltpu.async_copy(bufb_v, b_hbm.at[pl.ds(hb, _HALF)], sem1)
        wa.wait()
        wb.wait()


@functools.lru_cache(maxsize=None)
def _sc_kernels():
    mesh = plsc.VectorSubcoreMesh(core_axis_name="c", subcore_axis_name="s",
                                  num_cores=_NC, num_subcores=_NS)
    dispatch = pl.kernel(
        _dispatch_sc_body,
        out_type=jax.ShapeDtypeStruct((P, H), jnp.float32),
        mesh=mesh,
        scratch_types=[
            pltpu.VMEM((CHUNK,), jnp.int32),
            pltpu.VMEM((CHUNK,), jnp.int32),
            pltpu.VMEM((CHUNK, H), jnp.float32),
            pltpu.SemaphoreType.DMA,
            pltpu.SemaphoreType.DMA,
            pltpu.SemaphoreType.DMA,
        ],
    )
    gather = pl.kernel(
        _gather_sc_body,
        out_type=[jax.ShapeDtypeStruct((T, H), jnp.float32),
                  jax.ShapeDtypeStruct((T, H), jnp.float32)],
        mesh=mesh,
        scratch_types=[
            pltpu.VMEM((_HALF,), jnp.int32),
            pltpu.VMEM((_HALF,), jnp.int32),
            pltpu.VMEM((_HALF, H), jnp.float32),
            pltpu.VMEM((_HALF, H), jnp.float32),
            pltpu.SemaphoreType.DMA,
            pltpu.SemaphoreType.DMA,
        ],
    )
    return dispatch, gather


def kernel(hidden_states, router_w, router_b, expert_gate, expert_up,
           expert_down, shared_conv_w, shared_up, shared_down):
    x = hidden_states.reshape(T, H)
    x16 = x.astype(jnp.bfloat16)

    d0c, d1c, w0c, w1c, offs8, padded8 = pl.pallas_call(
        _router_body,
        out_shape=[
            jax.ShapeDtypeStruct((T, 1), jnp.int32),
            jax.ShapeDtypeStruct((T, 1), jnp.int32),
            jax.ShapeDtypeStruct((T, 1), jnp.float32),
            jax.ShapeDtypeStruct((T, 1), jnp.float32),
            jax.ShapeDtypeStruct((8, E), jnp.int32),
            jax.ShapeDtypeStruct((8, E), jnp.int32),
        ],
    )(x16, router_w.astype(jnp.bfloat16), router_b.reshape(1, E))

    d0 = d0c.reshape(T)
    d1 = d1c.reshape(T)

    offs = offs8[0]
    padded = padded8[0]
    total = offs[E - 1] + padded[E - 1]
    tile_start = jnp.arange(NT, dtype=jnp.int32) * TILE
    te_idx = jnp.sum(
        (tile_start[:, None] >= offs[None, :]).astype(jnp.int32), axis=1) - 1
    act = (tile_start < total).astype(jnp.int32)
    last_te = te_idx[jnp.maximum((total >> _TSH) - 1, 0)]
    te_fix = jnp.where(act == 1, te_idx, last_te).astype(jnp.int32)

    _dispatch_sc, _gather_sc = _sc_kernels()
    xg = _dispatch_sc(x, d0, d1)

    go = pl.pallas_call(
        _expert_body,
        grid_spec=pltpu.PrefetchScalarGridSpec(
            num_scalar_prefetch=2,
            grid=(NT,),
            in_specs=[
                pl.BlockSpec(
                    (TILE, H),
                    lambda i, te, a: (jnp.where(a[i] == 1, i, 0), 0)),
                pl.BlockSpec((1, H, DFF), lambda i, te, a: (te[i], 0, 0)),
                pl.BlockSpec((1, H, DFF), lambda i, te, a: (te[i], 0, 0)),
                pl.BlockSpec((1, DFF, H), lambda i, te, a: (te[i], 0, 0)),
            ],
            out_specs=pl.BlockSpec(
                (TILE, H),
                lambda i, te, a: (jnp.where(a[i] == 1, i, NT - 1), 0)),
        ),
        out_shape=jax.ShapeDtypeStruct((P, H), jnp.float32),
    )(te_fix, act, xg,
      expert_gate.astype(jnp.bfloat16),
      expert_up.astype(jnp.bfloat16),
      expert_down.astype(jnp.bfloat16))

    arows, brows = _gather_sc(go, d0, d1)

    xp = jnp.pad(x16, ((KSZ - 1, 0), (0, 0)))
    xcat = jnp.concatenate([xp[k:T + k] for k in range(KSZ)], axis=1)
    wcat = jnp.concatenate(
        [shared_conv_w[:, :, k].T for k in range(KSZ)], axis=0
    ).astype(jnp.bfloat16)
    TM = 512
    out = pl.pallas_call(
        _shared_body,
        grid=(T // TM,),
        in_specs=[
            pl.BlockSpec((TM, KSZ * H), lambda i: (i, 0)),
            pl.BlockSpec((TM, H), lambda i: (i, 0)),
            pl.BlockSpec((KSZ * H, DFF), lambda i: (0, 0)),
            pl.BlockSpec((H, DFF), lambda i: (0, 0)),
            pl.BlockSpec((DFF, H), lambda i: (0, 0)),
            pl.BlockSpec((TM, H), lambda i: (i, 0)),
            pl.BlockSpec((TM, H), lambda i: (i, 0)),
            pl.BlockSpec((TM, 1), lambda i: (i, 0)),
            pl.BlockSpec((TM, 1), lambda i: (i, 0)),
        ],
        out_specs=pl.BlockSpec((TM, H), lambda i: (i, 0)),
        out_shape=jax.ShapeDtypeStruct((T, H), jnp.float32),
    )(xcat, x16, wcat,
      shared_up.astype(jnp.bfloat16), shared_down.astype(jnp.bfloat16),
      arows, brows, w0c, w1c)
    return out.reshape(B, S, H)
